# baseline (speedup 1.0000x reference)
import functools

import jax
import jax.numpy as jnp
from jax import lax
from jax.experimental import pallas as pl
from jax.experimental.pallas import tpu as pltpu
from jax.experimental.pallas import tpu_sc as plsc

QL = 1000
D = 192
LAT = 32
B = 4096 * 50
NC, NS = 2, 16
NW = NC * NS
ROWS_PER_W = B // NW
CHUNK = 128
NSTEP = ROWS_PER_W // CHUNK


def _act_body(table_ref, qs_ref, dz_ref, rh_ref, th_ref, rs_ref, ts_ref):
    t = table_ref[...]
    qs_ref[...] = jax.nn.softplus(t[:, 0 * LAT:1 * LAT])
    dz_ref[...] = jax.nn.softplus(t[:, 1 * LAT:2 * LAT])
    rh_ref[...] = jax.nn.sigmoid(t[:, 2 * LAT:3 * LAT])
    th_ref[...] = jax.nn.sigmoid(t[:, 3 * LAT:4 * LAT])
    rs_ref[...] = jax.nn.sigmoid(t[:, 4 * LAT:5 * LAT])
    ts_ref[...] = jax.nn.sigmoid(t[:, 5 * LAT:6 * LAT])


_act_tables = pl.pallas_call(
    _act_body,
    out_shape=tuple(
        jax.ShapeDtypeStruct((QL, LAT), jnp.float32) for _ in range(6)
    ),
)


def _sc_gather(ids_hbm, table_hbm, t0, t1, t2, t3, t4, t5,
               x_out, o0, o1, o2, o3, o4, o5,
               idx_v, rows_v, b0, b1, b2, b3, b4, b5, sem):
    wid = lax.axis_index("s") * NC + lax.axis_index("c")
    base = wid * ROWS_PER_W
    acts = ((t0, b0, o0), (t1, b1, o1), (t2, b2, o2),
            (t3, b3, o3), (t4, b4, o4), (t5, b5, o5))

    def step(i, carry):
        off = base + i * CHUNK
        pltpu.sync_copy(ids_hbm.at[pl.ds(off, CHUNK)], idx_v)
        pltpu.async_copy(table_hbm.at[idx_v], rows_v, sem).wait()
        pltpu.sync_copy(rows_v, x_out.at[pl.ds(off, CHUNK)])
        for t, b, o in acts:
            pltpu.async_copy(t.at[idx_v], b, sem).wait()
            pltpu.sync_copy(b, o.at[pl.ds(off, CHUNK)])
        return carry

    lax.fori_loop(0, NSTEP, step, 0)


_gather = functools.partial(
    pl.kernel,
    out_type=(
        jax.ShapeDtypeStruct((B, D), jnp.float32),
        *(jax.ShapeDtypeStruct((B, LAT), jnp.float32) for _ in range(6)),
    ),
    mesh=plsc.VectorSubcoreMesh(core_axis_name="c", subcore_axis_name="s"),
    compiler_params=pltpu.CompilerParams(use_tc_tiling_on_sc=False),
    scratch_types=[
        pltpu.VMEM((CHUNK,), jnp.int32),
        pltpu.VMEM((CHUNK, D), jnp.float32),
        *(pltpu.VMEM((CHUNK, LAT), jnp.float32) for _ in range(6)),
        pltpu.SemaphoreType.DMA,
    ],
)(_sc_gather)


def kernel(quant_ids, table):
    ids = quant_ids.reshape(-1)
    acts = _act_tables(table)
    x, qs, dz, rh, th, rs, ts = _gather(ids, table, *acts)
    s = quant_ids.shape
    return (
        x.reshape(*s, D),
        qs.reshape(*s, LAT),
        dz.reshape(*s, LAT),
        rh.reshape(*s, LAT),
        th.reshape(*s, LAT),
        rs.reshape(*s, LAT),
        ts.reshape(*s, LAT),
    )
